# baseline (speedup 1.0000x reference)
import jax
import jax.numpy as jnp
from jax.experimental import pallas as pl
from jax.experimental.pallas import tpu as pltpu

_NBUF = 2
_CHUNKS_PER_REL = 2


def _compgcn_kernel(adj_ref, x_ref, rel_ref, ws_ref, wl_ref, wr_ref,
                    b_ref, lr_ref, out_ref, abuf, mcat_ref, sems):
    n = x_ref.shape[0]
    r_count = adj_ref.shape[0]
    num_layers = ws_ref.shape[0]
    f32 = jnp.float32
    bf16 = jnp.bfloat16
    dn = (((1,), (0,)), ((), ()))
    dn_t = (((0,), (0,)), ((), ()))
    rows = n // _CHUNKS_PER_REL
    nchunks = r_count * _CHUNKS_PER_REL

    def copy(c, slot):
        et, cc = divmod(c, _CHUNKS_PER_REL)
        return pltpu.make_async_copy(
            adj_ref.at[et, pl.ds(cc * rows, rows), :],
            abuf.at[slot], sems.at[slot])

    for c in range(_NBUF):
        copy(c, c).start()
    ones_row = jnp.ones((1, rows), bf16)
    deg = jnp.zeros((1, n), f32)
    for c in range(nchunks):
        slot = c % _NBUF
        copy(c, slot).wait()
        m = (abuf[slot] > 0.5).astype(bf16)
        nxt = c + _NBUF
        if nxt < nchunks:
            copy(nxt, slot).start()
        mcat_ref[pl.ds(c * rows, rows), :] = m
        deg = deg + jax.lax.dot_general(ones_row, m, dn,
                                        preferred_element_type=f32)

    ht = x_ref[...].T
    rt = rel_ref[0:r_count, :].T
    bt = b_ref[...].T
    lrt = jnp.concatenate([lr_ref[l] for l in range(num_layers)], axis=0).T

    norm = jnp.where(deg > 0.0, jax.lax.rsqrt(deg), 0.0)
    normcat = jnp.concatenate([norm] * r_count, axis=1)
    mcat = mcat_ref[...]

    for l in range(num_layers):
        comp = jnp.concatenate(
            [ht * rt[:, et:et + 1] for et in range(r_count)], axis=1)
        compw = jax.lax.dot_general(ws_ref[l], comp, dn_t)
        compw = (compw * normcat).astype(bf16)
        aggt = jax.lax.dot_general(compw, mcat, dn,
                                   preferred_element_type=f32) * norm
        loopt = jax.lax.dot_general(wl_ref[l], ht * lrt[:, l:l + 1], dn_t)
        ht = jnp.tanh(aggt + loopt + bt[:, l:l + 1])
        if l + 1 < num_layers:
            rt = jax.lax.dot_general(wr_ref[l], rt, dn_t)
    out_ref[...] = ht.T


@jax.jit
def kernel(x, fw_adjs, init_rel, Ws, W_loops, W_rels, biases, loop_rels):
    n, h_dim = x.shape
    r_count = fw_adjs.shape[0]
    rows = n // _CHUNKS_PER_REL
    vmem = pltpu.MemorySpace.VMEM
    return pl.pallas_call(
        _compgcn_kernel,
        in_specs=[
            pl.BlockSpec(memory_space=pl.ANY),
            pl.BlockSpec(memory_space=vmem),
            pl.BlockSpec(memory_space=vmem),
            pl.BlockSpec(memory_space=vmem),
            pl.BlockSpec(memory_space=vmem),
            pl.BlockSpec(memory_space=vmem),
            pl.BlockSpec(memory_space=vmem),
            pl.BlockSpec(memory_space=vmem),
        ],
        out_specs=pl.BlockSpec(memory_space=vmem),
        out_shape=jax.ShapeDtypeStruct((n, h_dim), x.dtype),
        scratch_shapes=[
            pltpu.VMEM((_NBUF, rows, n), jnp.float32),
            pltpu.VMEM((r_count * n, n), jnp.bfloat16),
            pltpu.SemaphoreType.DMA((_NBUF,)),
        ],
    )(fw_adjs, x, init_rel, Ws, W_loops, W_rels, biases, loop_rels)


# speedup vs baseline: 1.4202x; 1.4202x over previous
import jax
import jax.numpy as jnp
from jax.experimental import pallas as pl


def _compgcn_kernel(adj_ref, x_ref, rel_ref, ws_ref, wl_ref, wr_ref,
                    b_ref, lr_ref, out_ref):
    n = x_ref.shape[0]
    r_count = adj_ref.shape[0]
    num_layers = ws_ref.shape[0]
    f32 = jnp.float32
    bf16 = jnp.bfloat16
    dn = (((1,), (0,)), ((), ()))
    dn_t = (((0,), (0,)), ((), ()))

    mcat = jnp.concatenate(
        [(adj_ref[et] > 0.5).astype(bf16) for et in range(r_count)], axis=0)
    deg = jax.lax.dot_general(jnp.ones((1, r_count * n), bf16), mcat, dn,
                              preferred_element_type=f32)
    norm = jnp.where(deg > 0.0, jax.lax.rsqrt(deg), 0.0)
    normcat = jnp.concatenate([norm] * r_count, axis=1)

    ht = x_ref[...].T
    rt = rel_ref[0:r_count, :].T
    bt = b_ref[...].T
    lrt = jnp.concatenate([lr_ref[l] for l in range(num_layers)], axis=0).T
    for l in range(num_layers):
        comp = jnp.concatenate(
            [ht * rt[:, et:et + 1] for et in range(r_count)], axis=1)
        compw = jax.lax.dot_general(ws_ref[l], comp, dn_t)
        compw = (compw * normcat).astype(bf16)
        aggt = jax.lax.dot_general(compw, mcat, dn,
                                   preferred_element_type=f32) * norm
        loopt = jax.lax.dot_general(wl_ref[l], ht * lrt[:, l:l + 1], dn_t)
        ht = jnp.tanh(aggt + loopt + bt[:, l:l + 1])
        if l + 1 < num_layers:
            rt = jax.lax.dot_general(wr_ref[l], rt, dn_t)
    out_ref[...] = ht.T


@jax.jit
def kernel(x, fw_adjs, init_rel, Ws, W_loops, W_rels, biases, loop_rels):
    n, h_dim = x.shape
    return pl.pallas_call(
        _compgcn_kernel,
        out_shape=jax.ShapeDtypeStruct((n, h_dim), x.dtype),
    )(fw_adjs, x, init_rel, Ws, W_loops, W_rels, biases, loop_rels)
